# baseline (speedup 1.0000x reference)
import functools

import jax
import jax.numpy as jnp
from jax.experimental import pallas as pl
from jax.experimental.pallas import tpu as pltpu

_G = 64
_BLK = 1000


def _fused_body(nblk, x1_ref, bat1_hbm, x2_ref, bat2_hbm, w1_ref, b1_ref,
                w2_ref, b2_ref, fcw_ref, fcb_ref, out_ref,
                s1_ref, c1_ref, s2_ref, c2_ref, st1_ref, st2_ref,
                bat1_vm, bat2_vm, sem1, sem2):
    i = pl.program_id(0)
    blk = x1_ref.shape[0]
    n = bat1_vm.shape[0]
    gids = jax.lax.broadcasted_iota(jnp.int32, (_G, 1), 0)

    @pl.when(i == 0)
    def _init():
        cp1 = pltpu.make_async_copy(bat1_hbm, bat1_vm, sem1)
        cp2 = pltpu.make_async_copy(bat2_hbm, bat2_vm, sem2)
        cp1.start()
        cp2.start()
        s1_ref[...] = jnp.zeros_like(s1_ref)
        s2_ref[...] = jnp.zeros_like(s2_ref)
        tri = (jax.lax.broadcasted_iota(jnp.int32, (_G, _G), 1)
               < jax.lax.broadcasted_iota(jnp.int32, (_G, _G), 0)
               ).astype(jnp.float32)
        cp1.wait()
        cnt1 = jnp.sum((bat1_vm[...].reshape(1, n) == gids).astype(jnp.float32),
                       axis=1, keepdims=True)
        c1_ref[...] = cnt1
        st1_ref[...] = jnp.dot(tri, cnt1, preferred_element_type=jnp.float32)
        cp2.wait()
        cnt2 = jnp.sum((bat2_vm[...].reshape(1, n) == gids).astype(jnp.float32),
                       axis=1, keepdims=True)
        c2_ref[...] = cnt2
        st2_ref[...] = jnp.dot(tri, cnt2, preferred_element_type=jnp.float32)

    rows = (i * blk + jax.lax.broadcasted_iota(jnp.int32, (1, blk), 1)
            ).astype(jnp.float32)

    def accum(x_ref, w_ref, b_ref, s_ref, c_ref, st_ref):
        h = jnp.maximum(
            jnp.dot(x_ref[...], w_ref[...],
                    preferred_element_type=jnp.float32)
            + b_ref[...].reshape(1, -1), 0.0)
        start = st_ref[...]
        stop = start + c_ref[...]
        onehot_t = ((rows >= start) & (rows < stop)).astype(jnp.float32)
        s_ref[...] += jnp.dot(onehot_t, h, preferred_element_type=jnp.float32)

    accum(x1_ref, w1_ref, b1_ref, s1_ref, c1_ref, st1_ref)
    accum(x2_ref, w2_ref, b2_ref, s2_ref, c2_ref, st2_ref)

    @pl.when(i == nblk - 1)
    def _finish():
        p1 = s1_ref[...] / jnp.maximum(c1_ref[...], 1.0)
        p2 = s2_ref[...] / jnp.maximum(c2_ref[...], 1.0)
        pool = (p1 + p2) * 0.5
        out_ref[...] = (jnp.dot(pool, fcw_ref[...],
                                preferred_element_type=jnp.float32)
                        + fcb_ref[...].reshape(1, -1))


@jax.jit
def _run(x1, bat1, x2, bat2, W1, b1, W2, b2, fcW, fcb):
    n, f1 = x1.shape
    h = W1.shape[1]
    out_dim = fcW.shape[1]
    nblk = n // _BLK

    row_spec = pl.BlockSpec((_BLK, f1), lambda i: (i, 0))
    hbm_spec = pl.BlockSpec(memory_space=pl.ANY)
    full = lambda a: pl.BlockSpec(a.shape, lambda i: (0,) * a.ndim)

    return pl.pallas_call(
        functools.partial(_fused_body, nblk),
        grid=(nblk,),
        in_specs=[row_spec, hbm_spec, row_spec, hbm_spec,
                  full(W1), full(b1), full(W2), full(b2),
                  full(fcW), full(fcb)],
        out_specs=pl.BlockSpec((_G, out_dim), lambda i: (0, 0)),
        out_shape=jax.ShapeDtypeStruct((_G, out_dim), jnp.float32),
        scratch_shapes=[
            pltpu.VMEM((_G, h), jnp.float32),
            pltpu.VMEM((_G, 1), jnp.float32),
            pltpu.VMEM((_G, h), jnp.float32),
            pltpu.VMEM((_G, 1), jnp.float32),
            pltpu.VMEM((_G, 1), jnp.float32),
            pltpu.VMEM((_G, 1), jnp.float32),
            pltpu.VMEM((n,), jnp.int32),
            pltpu.VMEM((n,), jnp.int32),
            pltpu.SemaphoreType.DMA,
            pltpu.SemaphoreType.DMA,
        ],
    )(x1, bat1, x2, bat2, W1, b1, W2, b2, fcW, fcb)


def kernel(x1, edge_index1, edge_attr1, batch1, x2, edge_index2, edge_attr2,
           batch2, W1, b1, W2, b2, fcW, fcb):
    del edge_index1, edge_attr1, edge_index2, edge_attr2
    return _run(x1, batch1, x2, batch2, W1, b1, W2, b2, fcW, fcb)


# speedup vs baseline: 1.2119x; 1.2119x over previous
import functools

import jax
import jax.numpy as jnp
from jax.experimental import pallas as pl
from jax.experimental.pallas import tpu as pltpu

_G = 64
_BLK = 5000


def _fused_body(nblk, x1_ref, bat1_hbm, x2_ref, bat2_hbm, w1_ref, b1_ref,
                w2_ref, b2_ref, fcw_ref, fcb_ref, out_ref,
                s1_ref, c1_ref, s2_ref, c2_ref, st1_ref, st2_ref,
                bat1_vm, bat2_vm, sem1, sem2):
    i = pl.program_id(0)
    blk = x1_ref.shape[0]
    n = bat1_vm.shape[0]
    gids = jax.lax.broadcasted_iota(jnp.int32, (_G, 1), 0)

    @pl.when(i == 0)
    def _init():
        cp1 = pltpu.make_async_copy(bat1_hbm, bat1_vm, sem1)
        cp2 = pltpu.make_async_copy(bat2_hbm, bat2_vm, sem2)
        cp1.start()
        cp2.start()
        s1_ref[...] = jnp.zeros_like(s1_ref)
        s2_ref[...] = jnp.zeros_like(s2_ref)
        tri = (jax.lax.broadcasted_iota(jnp.int32, (_G, _G), 1)
               < jax.lax.broadcasted_iota(jnp.int32, (_G, _G), 0)
               ).astype(jnp.float32)
        cp1.wait()
        cnt1 = jnp.sum((bat1_vm[...].reshape(1, n) == gids).astype(jnp.float32),
                       axis=1, keepdims=True)
        c1_ref[...] = cnt1
        st1_ref[...] = jnp.dot(tri, cnt1, preferred_element_type=jnp.float32)
        cp2.wait()
        cnt2 = jnp.sum((bat2_vm[...].reshape(1, n) == gids).astype(jnp.float32),
                       axis=1, keepdims=True)
        c2_ref[...] = cnt2
        st2_ref[...] = jnp.dot(tri, cnt2, preferred_element_type=jnp.float32)

    rows = (i * blk + jax.lax.broadcasted_iota(jnp.int32, (1, blk), 1)
            ).astype(jnp.float32)

    def accum(x_ref, w_ref, b_ref, s_ref, c_ref, st_ref):
        h = jnp.maximum(
            jnp.dot(x_ref[...], w_ref[...],
                    preferred_element_type=jnp.float32)
            + b_ref[...].reshape(1, -1), 0.0)
        start = st_ref[...]
        stop = start + c_ref[...]
        onehot_t = ((rows >= start) & (rows < stop)).astype(jnp.float32)
        s_ref[...] += jnp.dot(onehot_t, h, preferred_element_type=jnp.float32)

    accum(x1_ref, w1_ref, b1_ref, s1_ref, c1_ref, st1_ref)
    accum(x2_ref, w2_ref, b2_ref, s2_ref, c2_ref, st2_ref)

    @pl.when(i == nblk - 1)
    def _finish():
        p1 = s1_ref[...] / jnp.maximum(c1_ref[...], 1.0)
        p2 = s2_ref[...] / jnp.maximum(c2_ref[...], 1.0)
        pool = (p1 + p2) * 0.5
        out_ref[...] = (jnp.dot(pool, fcw_ref[...],
                                preferred_element_type=jnp.float32)
                        + fcb_ref[...].reshape(1, -1))


@jax.jit
def _run(x1, bat1, x2, bat2, W1, b1, W2, b2, fcW, fcb):
    n, f1 = x1.shape
    h = W1.shape[1]
    out_dim = fcW.shape[1]
    nblk = n // _BLK

    row_spec = pl.BlockSpec((_BLK, f1), lambda i: (i, 0))
    hbm_spec = pl.BlockSpec(memory_space=pl.ANY)
    full = lambda a: pl.BlockSpec(a.shape, lambda i: (0,) * a.ndim)

    return pl.pallas_call(
        functools.partial(_fused_body, nblk),
        grid=(nblk,),
        in_specs=[row_spec, hbm_spec, row_spec, hbm_spec,
                  full(W1), full(b1), full(W2), full(b2),
                  full(fcW), full(fcb)],
        out_specs=pl.BlockSpec((_G, out_dim), lambda i: (0, 0)),
        out_shape=jax.ShapeDtypeStruct((_G, out_dim), jnp.float32),
        scratch_shapes=[
            pltpu.VMEM((_G, h), jnp.float32),
            pltpu.VMEM((_G, 1), jnp.float32),
            pltpu.VMEM((_G, h), jnp.float32),
            pltpu.VMEM((_G, 1), jnp.float32),
            pltpu.VMEM((_G, 1), jnp.float32),
            pltpu.VMEM((_G, 1), jnp.float32),
            pltpu.VMEM((n,), jnp.int32),
            pltpu.VMEM((n,), jnp.int32),
            pltpu.SemaphoreType.DMA,
            pltpu.SemaphoreType.DMA,
        ],
    )(x1, bat1, x2, bat2, W1, b1, W2, b2, fcW, fcb)


def kernel(x1, edge_index1, edge_attr1, batch1, x2, edge_index2, edge_attr2,
           batch2, W1, b1, W2, b2, fcW, fcb):
    del edge_index1, edge_attr1, edge_index2, edge_attr2
    return _run(x1, batch1, x2, batch2, W1, b1, W2, b2, fcW, fcb)


# speedup vs baseline: 1.3830x; 1.1411x over previous
import functools

import jax
import jax.numpy as jnp
from jax.experimental import pallas as pl
from jax.experimental.pallas import tpu as pltpu

_G = 64
_BLK = 2000


def _fused_body(nblk, x1_ref, bat1_hbm, x2_ref, bat2_hbm, w1_ref, b1_ref,
                w2_ref, b2_ref, fcw_ref, fcb_ref, out_ref,
                s1_ref, c1_ref, s2_ref, c2_ref, st1_ref, st2_ref,
                wcat_ref, bat1_vm, bat2_vm, sem1, sem2):
    i = pl.program_id(0)
    blk = x1_ref.shape[0]
    n = bat1_vm.shape[0]
    f = w1_ref.shape[0]
    h = w1_ref.shape[1]
    gids = jax.lax.broadcasted_iota(jnp.int32, (_G, 1), 0)

    @pl.when(i == 0)
    def _init():
        cp1 = pltpu.make_async_copy(bat1_hbm, bat1_vm, sem1)
        cp2 = pltpu.make_async_copy(bat2_hbm, bat2_vm, sem2)
        cp1.start()
        cp2.start()
        s1_ref[...] = jnp.zeros_like(s1_ref)
        s2_ref[...] = jnp.zeros_like(s2_ref)
        wcat_ref[...] = jnp.zeros_like(wcat_ref)
        wcat_ref[:f, :h] = w1_ref[...]
        wcat_ref[f:, h:] = w2_ref[...]
        tri = (jax.lax.broadcasted_iota(jnp.int32, (_G, _G), 1)
               < jax.lax.broadcasted_iota(jnp.int32, (_G, _G), 0)
               ).astype(jnp.float32)
        cp1.wait()
        cnt1 = jnp.sum((bat1_vm[...].reshape(1, n) == gids).astype(jnp.float32),
                       axis=1, keepdims=True)
        c1_ref[...] = cnt1
        st1_ref[...] = jnp.dot(tri, cnt1, preferred_element_type=jnp.float32)
        cp2.wait()
        cnt2 = jnp.sum((bat2_vm[...].reshape(1, n) == gids).astype(jnp.float32),
                       axis=1, keepdims=True)
        c2_ref[...] = cnt2
        st2_ref[...] = jnp.dot(tri, cnt2, preferred_element_type=jnp.float32)

    rows = (i * blk + jax.lax.broadcasted_iota(jnp.int32, (1, blk), 1)
            ).astype(jnp.float32)

    xcat = jnp.concatenate([x1_ref[...], x2_ref[...]], axis=1)
    bcat = jnp.concatenate([b1_ref[...].reshape(1, -1),
                            b2_ref[...].reshape(1, -1)], axis=1)
    hcat = jnp.maximum(
        jnp.dot(xcat, wcat_ref[...], preferred_element_type=jnp.float32)
        + bcat, 0.0)

    def accum(hpart, s_ref, c_ref, st_ref):
        start = st_ref[...]
        stop = start + c_ref[...]
        onehot_t = ((rows >= start) & (rows < stop)).astype(jnp.float32)
        s_ref[...] += jnp.dot(onehot_t, hpart,
                              preferred_element_type=jnp.float32)

    accum(hcat[:, :h], s1_ref, c1_ref, st1_ref)
    accum(hcat[:, h:], s2_ref, c2_ref, st2_ref)

    @pl.when(i == nblk - 1)
    def _finish():
        p1 = s1_ref[...] / jnp.maximum(c1_ref[...], 1.0)
        p2 = s2_ref[...] / jnp.maximum(c2_ref[...], 1.0)
        pool = (p1 + p2) * 0.5
        out_ref[...] = (jnp.dot(pool, fcw_ref[...],
                                preferred_element_type=jnp.float32)
                        + fcb_ref[...].reshape(1, -1))


@jax.jit
def _run(x1, bat1, x2, bat2, W1, b1, W2, b2, fcW, fcb):
    n, f1 = x1.shape
    h = W1.shape[1]
    out_dim = fcW.shape[1]
    nblk = n // _BLK

    row_spec = pl.BlockSpec((_BLK, f1), lambda i: (i, 0))
    hbm_spec = pl.BlockSpec(memory_space=pl.ANY)
    full = lambda a: pl.BlockSpec(a.shape, lambda i: (0,) * a.ndim)

    return pl.pallas_call(
        functools.partial(_fused_body, nblk),
        grid=(nblk,),
        in_specs=[row_spec, hbm_spec, row_spec, hbm_spec,
                  full(W1), full(b1), full(W2), full(b2),
                  full(fcW), full(fcb)],
        out_specs=pl.BlockSpec((_G, out_dim), lambda i: (0, 0)),
        out_shape=jax.ShapeDtypeStruct((_G, out_dim), jnp.float32),
        scratch_shapes=[
            pltpu.VMEM((_G, h), jnp.float32),
            pltpu.VMEM((_G, 1), jnp.float32),
            pltpu.VMEM((_G, h), jnp.float32),
            pltpu.VMEM((_G, 1), jnp.float32),
            pltpu.VMEM((_G, 1), jnp.float32),
            pltpu.VMEM((_G, 1), jnp.float32),
            pltpu.VMEM((2 * f1, 2 * h), jnp.float32),
            pltpu.VMEM((n,), jnp.int32),
            pltpu.VMEM((n,), jnp.int32),
            pltpu.SemaphoreType.DMA,
            pltpu.SemaphoreType.DMA,
        ],
    )(x1, bat1, x2, bat2, W1, b1, W2, b2, fcW, fcb)


def kernel(x1, edge_index1, edge_attr1, batch1, x2, edge_index2, edge_attr2,
           batch2, W1, b1, W2, b2, fcW, fcb):
    del edge_index1, edge_attr1, edge_index2, edge_attr2
    return _run(x1, batch1, x2, batch2, W1, b1, W2, b2, fcW, fcb)
